# Initial kernel scaffold; baseline (speedup 1.0000x reference)
import functools

import jax
import jax.numpy as jnp
from jax import lax
from jax.experimental import pallas as pl
from jax.experimental.pallas import tpu as pltpu
from jax.experimental.pallas import tpu_sc as plsc

NC = 2
NS = 16
LANES = 16


def _build_sc_agg(N, E, D, C):
    NW = NC * NS
    epw = E // NW
    nchunk = epw // C
    rpt = N // NS
    vchunks = (C * D) // LANES

    mesh = plsc.VectorSubcoreMesh(core_axis_name="c", subcore_axis_name="s")

    @functools.partial(
        pl.kernel,
        out_type=jax.ShapeDtypeStruct((NC, N, D), jnp.float32),
        mesh=mesh,
        scratch_types=[
            pltpu.VMEM((C,), jnp.int32),
            pltpu.VMEM((C,), jnp.int32),
            pltpu.VMEM((C, D), jnp.float32),
            pltpu.VMEM((C, D), jnp.float32),
            pltpu.VMEM_SHARED((N, D), jnp.float32),
            pltpu.SemaphoreType.DMA,
        ],
    )
    def sc_agg(x_hbm, src_hbm, dst_hbm, ea_hbm, out_hbm,
               src_v, dst_v, xr_v, ea_v, acc_sh, sem):
        cid = lax.axis_index("c")
        sid = lax.axis_index("s")
        wid = cid * NS + sid

        zero = jnp.zeros((LANES,), jnp.float32)

        def zrow(r, carry):
            for j in range(D // LANES):
                xr_v[r, pl.ds(j * LANES, LANES)] = zero
            return carry

        lax.fori_loop(0, C, zrow, 0)
        base = sid * rpt
        nfull = rpt // C
        rem = rpt - nfull * C
        for k in range(nfull):
            pltpu.sync_copy(xr_v, acc_sh.at[pl.ds(base + k * C, C)])
        if rem:
            pltpu.sync_copy(xr_v.at[pl.ds(0, rem)],
                            acc_sh.at[pl.ds(base + nfull * C, rem)])
        plsc.subcore_barrier()

        ebase = wid * epw

        def chunk(i, carry):
            off = ebase + i * C
            pltpu.sync_copy(src_hbm.at[pl.ds(off, C)], src_v)
            pltpu.sync_copy(dst_hbm.at[pl.ds(off, C)], dst_v)
            gat = pltpu.async_copy(x_hbm.at[src_v], xr_v, sem)
            pltpu.sync_copy(ea_hbm.at[pl.ds(off, C)], ea_v)
            gat.wait()

            def vrow(r, c2):
                row = r // (D // LANES)
                col = (r % (D // LANES)) * LANES
                s = pl.ds(col, LANES)
                ea_v[row, s] = jnp.maximum(ea_v[row, s] + xr_v[row, s], 0.0)
                return c2

            lax.fori_loop(0, vchunks, vrow, 0)
            pltpu.sync_copy(ea_v, acc_sh.at[dst_v], add=True)
            return carry

        lax.fori_loop(0, nchunk, chunk, 0)
        plsc.subcore_barrier()

        pltpu.sync_copy(acc_sh.at[pl.ds(sid * rpt, rpt)],
                        out_hbm.at[cid, pl.ds(sid * rpt, rpt)])

    return sc_agg


def _tc_ffn_body(x_ref, a0_ref, a1_ref, w1_ref, b1_ref, w2_ref, b2_ref, o_ref):
    xb = x_ref[...]
    h = xb + a0_ref[...] + a1_ref[...]
    h = jnp.dot(h, w1_ref[...], preferred_element_type=jnp.float32) + b1_ref[...]
    h = jnp.maximum(h, 0.0)
    h = jnp.dot(h, w2_ref[...], preferred_element_type=jnp.float32) + b2_ref[...]
    o_ref[...] = xb + jnp.maximum(h, 0.0)


def _tc_ffn(x, a0, a1, w1t, b1, w2t, b2, rows):
    N, D = x.shape
    grid = (N // rows,)
    row_spec = pl.BlockSpec((rows, D), lambda i: (i, 0))
    full_spec = pl.BlockSpec((D, D), lambda i: (0, 0))
    vec_spec = pl.BlockSpec((1, D), lambda i: (0, 0))
    return pl.pallas_call(
        _tc_ffn_body,
        grid=grid,
        in_specs=[row_spec, row_spec, row_spec,
                  full_spec, vec_spec, full_spec, vec_spec],
        out_specs=row_spec,
        out_shape=jax.ShapeDtypeStruct((N, D), jnp.float32),
    )(x, a0, a1, w1t, b1, w2t, b2)


def kernel(x, edge_index, edge_attr, W1, b1, W2, b2):
    N, D = x.shape
    E = edge_index.shape[1]
    src = edge_index[0]
    dst = edge_index[1]
    sc_agg = _build_sc_agg(N, E, D, C=80)
    aggs = sc_agg(x, src, dst, edge_attr)
    return _tc_ffn(x, aggs[0], aggs[1], W1.T, b1.reshape(1, D),
                   W2.T, b2.reshape(1, D), rows=400)


# speedup vs baseline: 2.2576x; 2.2576x over previous
import functools

import jax
import jax.numpy as jnp
from jax import lax
from jax.experimental import pallas as pl
from jax.experimental.pallas import tpu as pltpu
from jax.experimental.pallas import tpu_sc as plsc

NC = 2
NS = 16
LANES = 16


def _build_sc_agg(N, E, D, C):
    NW = NC * NS
    epw = E // NW
    nchunk = epw // C
    rpt = (-(-N // NS) + 7) // 8 * 8
    n_pad = NS * rpt
    vchunks = (C * D) // LANES

    mesh = plsc.VectorSubcoreMesh(core_axis_name="c", subcore_axis_name="s")

    @functools.partial(
        pl.kernel,
        out_type=jax.ShapeDtypeStruct((NC, n_pad, D), jnp.float32),
        mesh=mesh,
        scratch_types=[
            pltpu.VMEM((C,), jnp.int32),
            pltpu.VMEM((C,), jnp.int32),
            pltpu.VMEM((C, D), jnp.float32),
            pltpu.VMEM((C, D), jnp.float32),
            pltpu.VMEM_SHARED((n_pad, D), jnp.float32),
            pltpu.SemaphoreType.DMA,
        ],
    )
    def sc_agg(x_hbm, src_hbm, dst_hbm, ea_hbm, out_hbm,
               src_v, dst_v, xr_v, ea_v, acc_sh, sem):
        cid = lax.axis_index("c")
        sid = lax.axis_index("s")
        wid = cid * NS + sid

        zero = jnp.zeros((LANES,), jnp.float32)

        def zrow(r, carry):
            for j in range(D // LANES):
                xr_v[r, pl.ds(j * LANES, LANES)] = zero
            return carry

        lax.fori_loop(0, C, zrow, 0)
        base = sid * rpt
        nfull = rpt // C
        rem = rpt - nfull * C
        for k in range(nfull):
            pltpu.sync_copy(xr_v, acc_sh.at[pl.ds(base + k * C, C)])
        if rem:
            pltpu.sync_copy(xr_v.at[pl.ds(0, rem)],
                            acc_sh.at[pl.ds(base + nfull * C, rem)])
        plsc.subcore_barrier()

        ebase = wid * epw

        def chunk(i, carry):
            off = ebase + i * C
            pltpu.sync_copy(src_hbm.at[pl.ds(off, C)], src_v)
            pltpu.sync_copy(dst_hbm.at[pl.ds(off, C)], dst_v)
            gat = pltpu.async_copy(x_hbm.at[src_v], xr_v, sem)
            pltpu.sync_copy(ea_hbm.at[pl.ds(off, C)], ea_v)
            gat.wait()

            def vrow(r, c2):
                row = r // (D // LANES)
                col = (r % (D // LANES)) * LANES
                s = pl.ds(col, LANES)
                ea_v[row, s] = jnp.maximum(ea_v[row, s] + xr_v[row, s], 0.0)
                return c2

            lax.fori_loop(0, vchunks, vrow, 0)
            pltpu.sync_copy(ea_v, acc_sh.at[dst_v], add=True)
            return carry

        lax.fori_loop(0, nchunk, chunk, 0)
        plsc.subcore_barrier()

        pltpu.sync_copy(acc_sh.at[pl.ds(sid * rpt, rpt)],
                        out_hbm.at[cid, pl.ds(sid * rpt, rpt)])

    return sc_agg


def _tc_ffn_body(x_ref, a0_ref, a1_ref, w1_ref, b1_ref, w2_ref, b2_ref, o_ref):
    xb = x_ref[...]
    h = xb + a0_ref[...] + a1_ref[...]
    h = jnp.dot(h, w1_ref[...], preferred_element_type=jnp.float32) + b1_ref[...]
    h = jnp.maximum(h, 0.0)
    h = jnp.dot(h, w2_ref[...], preferred_element_type=jnp.float32) + b2_ref[...]
    o_ref[...] = xb + jnp.maximum(h, 0.0)


def _tc_ffn(x, a0, a1, w1t, b1, w2t, b2, rows):
    N, D = x.shape
    grid = (N // rows,)
    row_spec = pl.BlockSpec((rows, D), lambda i: (i, 0))
    agg_spec = pl.BlockSpec((rows, D), lambda i: (i, 0))
    full_spec = pl.BlockSpec((D, D), lambda i: (0, 0))
    vec_spec = pl.BlockSpec((1, D), lambda i: (0, 0))
    return pl.pallas_call(
        _tc_ffn_body,
        grid=grid,
        in_specs=[row_spec, agg_spec, agg_spec,
                  full_spec, vec_spec, full_spec, vec_spec],
        out_specs=row_spec,
        out_shape=jax.ShapeDtypeStruct((N, D), jnp.float32),
    )(x, a0, a1, w1t, b1, w2t, b2)


def kernel(x, edge_index, edge_attr, W1, b1, W2, b2):
    N, D = x.shape
    E = edge_index.shape[1]
    src = edge_index[0]
    dst = edge_index[1]
    sc_agg = _build_sc_agg(N, E, D, C=80)
    aggs = sc_agg(x, src, dst, edge_attr)
    return _tc_ffn(x, aggs[0], aggs[1], W1.T, b1.reshape(1, D),
                   W2.T, b2.reshape(1, D), rows=400)


# speedup vs baseline: 4.8280x; 2.1385x over previous
import functools

import jax
import jax.numpy as jnp
from jax import lax
from jax.experimental import pallas as pl
from jax.experimental.pallas import tpu as pltpu
from jax.experimental.pallas import tpu_sc as plsc

NC = 2
NS = 16
LANES = 16
K = 2


def _build_sc_agg(N, E, D, C):
    NW = NC * NS
    epw = E // NW
    nchunk = epw // C
    niter = nchunk // K
    assert nchunk == niter * K and niter >= 2
    rpt = (-(-N // NS) + 7) // 8 * 8
    n_pad = NS * rpt

    mesh = plsc.VectorSubcoreMesh(core_axis_name="c", subcore_axis_name="s")

    @functools.partial(
        pl.kernel,
        out_type=jax.ShapeDtypeStruct((NC, n_pad, D), jnp.float32),
        mesh=mesh,
        scratch_types=[
            pltpu.VMEM((K, C), jnp.int32),
            pltpu.VMEM((K, C), jnp.int32),
            pltpu.VMEM((K, C, D), jnp.float32),
            pltpu.VMEM((K, C, D), jnp.float32),
            pltpu.VMEM_SHARED((n_pad, D), jnp.float32),
            pltpu.SemaphoreType.DMA,
            pltpu.SemaphoreType.DMA,
            pltpu.SemaphoreType.DMA,
            pltpu.SemaphoreType.DMA,
            pltpu.SemaphoreType.DMA,
            pltpu.SemaphoreType.DMA,
            pltpu.SemaphoreType.DMA,
            pltpu.SemaphoreType.DMA,
        ],
    )
    def sc_agg(x_hbm, src_hbm, dst_hbm, ea_hbm, out_hbm,
               src_v, dst_v, xr_v, ea_v, acc_sh,
               si0, si1, sd0, sd1, se0, se1, sg0, sg1):
        si = (si0, si1)
        sd = (sd0, sd1)
        se = (se0, se1)
        sg = (sg0, sg1)
        cid = lax.axis_index("c")
        sid = lax.axis_index("s")
        wid = cid * NS + sid
        ebase = wid * epw

        def start_idx(c, j):
            off = ebase + c * C
            pltpu.async_copy(src_hbm.at[pl.ds(off, C)], src_v.at[j], si[j])
            pltpu.async_copy(dst_hbm.at[pl.ds(off, C)], dst_v.at[j], sd[j])

        def wait_idx(c, j):
            off = ebase + c * C
            pltpu.make_async_copy(src_hbm.at[pl.ds(off, C)], src_v.at[j],
                                  si[j]).wait()
            pltpu.make_async_copy(dst_hbm.at[pl.ds(off, C)], dst_v.at[j],
                                  sd[j]).wait()

        def start_ea(c, j):
            off = ebase + c * C
            pltpu.async_copy(ea_hbm.at[pl.ds(off, C)], ea_v.at[j], se[j])

        def wait_ea(c, j):
            off = ebase + c * C
            pltpu.make_async_copy(ea_hbm.at[pl.ds(off, C)], ea_v.at[j],
                                  se[j]).wait()

        def compute(j):
            def vrow(r, carry):
                for q in range(D // LANES):
                    s = pl.ds(q * LANES, LANES)
                    ea_v[j, r, s] = jnp.maximum(ea_v[j, r, s] + xr_v[j, r, s],
                                                0.0)
                return carry
            lax.fori_loop(0, C, vrow, 0)

        zero = jnp.zeros((LANES,), jnp.float32)

        def zrow(r, carry):
            for q in range(D // LANES):
                xr_v[0, r, pl.ds(q * LANES, LANES)] = zero
            return carry

        lax.fori_loop(0, C, zrow, 0)
        base = sid * rpt
        nfull = rpt // C
        rem = rpt - nfull * C
        for k in range(nfull):
            pltpu.sync_copy(xr_v.at[0], acc_sh.at[pl.ds(base + k * C, C)])
        if rem:
            pltpu.sync_copy(xr_v.at[0].at[pl.ds(0, rem)],
                            acc_sh.at[pl.ds(base + nfull * C, rem)])
        plsc.subcore_barrier()

        def iter_body(c0, prefetch):
            for j in range(K):
                wait_idx(c0 + j, j)
            gds = [pltpu.async_copy(x_hbm.at[src_v.at[j]], xr_v.at[j], sg[j])
                   for j in range(K)]
            for j in range(K):
                wait_ea(c0 + j, j)
                gds[j].wait()
                compute(j)
                pltpu.sync_copy(ea_v.at[j], acc_sh.at[dst_v.at[j]], add=True)
                if prefetch:
                    start_idx(c0 + K + j, j)
                    start_ea(c0 + K + j, j)

        for j in range(K):
            start_idx(j, j)
            start_ea(j, j)
        def loop_body(i, carry):
            iter_body(i * K, True)
            return carry
        lax.fori_loop(0, niter - 1, loop_body, 0)
        iter_body((niter - 1) * K, False)

        plsc.subcore_barrier()

        pltpu.sync_copy(acc_sh.at[pl.ds(sid * rpt, rpt)],
                        out_hbm.at[cid, pl.ds(sid * rpt, rpt)])

    return sc_agg


def _tc_ffn_body(x_ref, a0_ref, a1_ref, w1_ref, b1_ref, w2_ref, b2_ref, o_ref):
    xb = x_ref[...]
    h = xb + a0_ref[...] + a1_ref[...]
    h = jnp.dot(h, w1_ref[...], preferred_element_type=jnp.float32) + b1_ref[...]
    h = jnp.maximum(h, 0.0)
    h = jnp.dot(h, w2_ref[...], preferred_element_type=jnp.float32) + b2_ref[...]
    o_ref[...] = xb + jnp.maximum(h, 0.0)


def _tc_ffn(x, a0, a1, w1t, b1, w2t, b2, rows):
    N, D = x.shape
    grid = (N // rows,)
    row_spec = pl.BlockSpec((rows, D), lambda i: (i, 0))
    agg_spec = pl.BlockSpec((rows, D), lambda i: (i, 0))
    full_spec = pl.BlockSpec((D, D), lambda i: (0, 0))
    vec_spec = pl.BlockSpec((1, D), lambda i: (0, 0))
    return pl.pallas_call(
        _tc_ffn_body,
        grid=grid,
        in_specs=[row_spec, agg_spec, agg_spec,
                  full_spec, vec_spec, full_spec, vec_spec],
        out_specs=row_spec,
        out_shape=jax.ShapeDtypeStruct((N, D), jnp.float32),
    )(x, a0, a1, w1t, b1, w2t, b2)


def kernel(x, edge_index, edge_attr, W1, b1, W2, b2):
    N, D = x.shape
    E = edge_index.shape[1]
    src = edge_index[0]
    dst = edge_index[1]
    sc_agg = _build_sc_agg(N, E, D, C=40)
    aggs = sc_agg(x, src, dst, edge_attr)
    return _tc_ffn(x, aggs[0], aggs[1], W1.T, b1.reshape(1, D),
                   W2.T, b2.reshape(1, D), rows=400)


# speedup vs baseline: 6.0168x; 1.2462x over previous
import functools

import jax
import jax.numpy as jnp
from jax import lax
from jax.experimental import pallas as pl
from jax.experimental.pallas import tpu as pltpu
from jax.experimental.pallas import tpu_sc as plsc

NC = 2
NS = 16
LANES = 16
K = 4


def _build_sc_agg(N, E, D, C):
    NW = NC * NS
    epw = E // NW
    nchunk = epw // C
    niter = nchunk // K
    ntail = nchunk - niter * K
    assert niter >= 2 and ntail < K
    rpt = (-(-N // NS) + 7) // 8 * 8
    n_pad = NS * rpt

    mesh = plsc.VectorSubcoreMesh(core_axis_name="c", subcore_axis_name="s")

    @functools.partial(
        pl.kernel,
        out_type=jax.ShapeDtypeStruct((NC, n_pad, D), jnp.float32),
        mesh=mesh,
        scratch_types=[
            pltpu.VMEM((K, C), jnp.int32),
            pltpu.VMEM((K, C), jnp.int32),
            pltpu.VMEM((K, C, D), jnp.float32),
            pltpu.VMEM((K, C, D), jnp.float32),
            pltpu.VMEM_SHARED((n_pad, D), jnp.float32),
            [pltpu.SemaphoreType.DMA] * K,
            [pltpu.SemaphoreType.DMA] * K,
            [pltpu.SemaphoreType.DMA] * K,
            [pltpu.SemaphoreType.DMA] * K,
            [pltpu.SemaphoreType.DMA] * K,
        ],
    )
    def sc_agg(x_hbm, src_hbm, dst_hbm, ea_hbm, out_hbm,
               src_v, dst_v, xr_v, ea_v, acc_sh,
               si, sd, se, sg, ssc):
        cid = lax.axis_index("c")
        sid = lax.axis_index("s")
        wid = cid * NS + sid
        ebase = wid * epw

        def start_idx(c, j):
            off = ebase + c * C
            pltpu.async_copy(src_hbm.at[pl.ds(off, C)], src_v.at[j], si[j])
            pltpu.async_copy(dst_hbm.at[pl.ds(off, C)], dst_v.at[j], sd[j])

        def wait_idx(c, j):
            off = ebase + c * C
            pltpu.make_async_copy(src_hbm.at[pl.ds(off, C)], src_v.at[j],
                                  si[j]).wait()
            pltpu.make_async_copy(dst_hbm.at[pl.ds(off, C)], dst_v.at[j],
                                  sd[j]).wait()

        def start_ea(c, j):
            off = ebase + c * C
            pltpu.async_copy(ea_hbm.at[pl.ds(off, C)], ea_v.at[j], se[j])

        def wait_ea(c, j):
            off = ebase + c * C
            pltpu.make_async_copy(ea_hbm.at[pl.ds(off, C)], ea_v.at[j],
                                  se[j]).wait()

        def compute(j):
            def vrow(r, carry):
                for u in range(2):
                    for q in range(D // LANES):
                        s = pl.ds(q * LANES, LANES)
                        ea_v[j, 2 * r + u, s] = jnp.maximum(
                            ea_v[j, 2 * r + u, s] + xr_v[j, 2 * r + u, s], 0.0)
                return carry
            lax.fori_loop(0, C // 2, vrow, 0)

        def iter_body(c0, js, prefetch_js):
            for j in js:
                wait_idx(c0 + j, j)
            gds = [pltpu.async_copy(x_hbm.at[src_v.at[j]], xr_v.at[j], sg[j])
                   for j in js]
            scs = []
            for n, j in enumerate(js):
                wait_ea(c0 + j, j)
                gds[n].wait()
                compute(j)
                scs.append(pltpu.async_copy(
                    ea_v.at[j], acc_sh.at[dst_v.at[j]], ssc[j], add=True))
            for n, j in enumerate(js):
                scs[n].wait()
                if j in prefetch_js:
                    start_idx(c0 + K + j, j)
                    start_ea(c0 + K + j, j)

        zero = jnp.zeros((LANES,), jnp.float32)

        def zrow(r, carry):
            for q in range(D // LANES):
                xr_v[0, r, pl.ds(q * LANES, LANES)] = zero
            return carry

        lax.fori_loop(0, C, zrow, 0)
        base = sid * rpt
        nfull = rpt // C
        rem = rpt - nfull * C
        for k in range(nfull):
            pltpu.sync_copy(xr_v.at[0], acc_sh.at[pl.ds(base + k * C, C)])
        if rem:
            pltpu.sync_copy(xr_v.at[0].at[pl.ds(0, rem)],
                            acc_sh.at[pl.ds(base + nfull * C, rem)])
        plsc.subcore_barrier()

        all_js = tuple(range(K))
        tail_js = tuple(range(ntail))
        for j in all_js:
            start_idx(j, j)
            start_ea(j, j)
        def loop_body(i, carry):
            iter_body(i * K, all_js, all_js)
            return carry
        lax.fori_loop(0, niter - 1, loop_body, 0)
        iter_body((niter - 1) * K, all_js, tail_js)
        if ntail:
            iter_body(niter * K, tail_js, ())

        plsc.subcore_barrier()

        pltpu.sync_copy(acc_sh.at[pl.ds(sid * rpt, rpt)],
                        out_hbm.at[cid, pl.ds(sid * rpt, rpt)])

    return sc_agg


def _tc_ffn_body(x_ref, a0_ref, a1_ref, w1_ref, b1_ref, w2_ref, b2_ref, o_ref):
    xb = x_ref[...]
    h = xb + a0_ref[...] + a1_ref[...]
    h = jnp.dot(h, w1_ref[...], preferred_element_type=jnp.float32) + b1_ref[...]
    h = jnp.maximum(h, 0.0)
    h = jnp.dot(h, w2_ref[...], preferred_element_type=jnp.float32) + b2_ref[...]
    o_ref[...] = xb + jnp.maximum(h, 0.0)


def _tc_ffn(x, a0, a1, w1t, b1, w2t, b2, rows):
    N, D = x.shape
    grid = (N // rows,)
    row_spec = pl.BlockSpec((rows, D), lambda i: (i, 0))
    agg_spec = pl.BlockSpec((rows, D), lambda i: (i, 0))
    full_spec = pl.BlockSpec((D, D), lambda i: (0, 0))
    vec_spec = pl.BlockSpec((1, D), lambda i: (0, 0))
    return pl.pallas_call(
        _tc_ffn_body,
        grid=grid,
        in_specs=[row_spec, agg_spec, agg_spec,
                  full_spec, vec_spec, full_spec, vec_spec],
        out_specs=row_spec,
        out_shape=jax.ShapeDtypeStruct((N, D), jnp.float32),
    )(x, a0, a1, w1t, b1, w2t, b2)


def kernel(x, edge_index, edge_attr, W1, b1, W2, b2):
    N, D = x.shape
    E = edge_index.shape[1]
    src = edge_index[0]
    dst = edge_index[1]
    sc_agg = _build_sc_agg(N, E, D, C=40)
    aggs = sc_agg(x, src, dst, edge_attr)
    return _tc_ffn(x, aggs[0], aggs[1], W1.T, b1.reshape(1, D),
                   W2.T, b2.reshape(1, D), rows=400)


# speedup vs baseline: 6.3233x; 1.0509x over previous
import functools

import jax
import jax.numpy as jnp
from jax import lax
from jax.experimental import pallas as pl
from jax.experimental.pallas import tpu as pltpu
from jax.experimental.pallas import tpu_sc as plsc

NC = 2
NS = 16
LANES = 16
K = 4


def _build_sc_agg(N, E, D, C):
    NW = NC * NS
    epw = E // NW
    nchunk = epw // C
    niter = nchunk // K
    ntail = nchunk - niter * K
    assert niter >= 2 and ntail < K
    rpt = (-(-N // NS) + 7) // 8 * 8
    n_pad = NS * rpt

    mesh = plsc.VectorSubcoreMesh(core_axis_name="c", subcore_axis_name="s")

    @functools.partial(
        pl.kernel,
        out_type=jax.ShapeDtypeStruct((NC, n_pad, D), jnp.float32),
        mesh=mesh,
        scratch_types=[
            pltpu.VMEM((K, C), jnp.int32),
            pltpu.VMEM((K, C), jnp.int32),
            pltpu.VMEM((K, C, D), jnp.float32),
            pltpu.VMEM((K, C, D), jnp.float32),
            pltpu.VMEM_SHARED((n_pad, D), jnp.float32),
            [pltpu.SemaphoreType.DMA] * K,
            [pltpu.SemaphoreType.DMA] * K,
            [pltpu.SemaphoreType.DMA] * K,
            [pltpu.SemaphoreType.DMA] * K,
            [pltpu.SemaphoreType.DMA] * K,
        ],
    )
    def sc_agg(x_hbm, src_hbm, dst_hbm, ea_hbm, out_hbm,
               src_v, dst_v, xr_v, ea_v, acc_sh,
               si, sd, se, sg, ssc):
        cid = lax.axis_index("c")
        sid = lax.axis_index("s")
        wid = cid * NS + sid
        ebase = wid * epw

        def start_idx(c, j):
            off = ebase + c * C
            pltpu.async_copy(src_hbm.at[pl.ds(off, C)], src_v.at[j], si[j])
            pltpu.async_copy(dst_hbm.at[pl.ds(off, C)], dst_v.at[j], sd[j])

        def wait_idx(c, j):
            off = ebase + c * C
            pltpu.make_async_copy(src_hbm.at[pl.ds(off, C)], src_v.at[j],
                                  si[j]).wait()
            pltpu.make_async_copy(dst_hbm.at[pl.ds(off, C)], dst_v.at[j],
                                  sd[j]).wait()

        def start_ea(c, j):
            off = ebase + c * C
            pltpu.async_copy(ea_hbm.at[pl.ds(off, C)], ea_v.at[j], se[j])

        def wait_ea(c, j):
            off = ebase + c * C
            pltpu.make_async_copy(ea_hbm.at[pl.ds(off, C)], ea_v.at[j],
                                  se[j]).wait()

        def compute(j):
            def vrow(r, carry):
                for u in range(2):
                    for q in range(D // LANES):
                        s = pl.ds(q * LANES, LANES)
                        ea_v[j, 2 * r + u, s] = jnp.maximum(
                            ea_v[j, 2 * r + u, s] + xr_v[j, 2 * r + u, s], 0.0)
                return carry
            lax.fori_loop(0, C // 2, vrow, 0)

        def iter_body(c0, js, prefetch_js):
            for j in js:
                wait_idx(c0 + j, j)
            gds = [pltpu.async_copy(x_hbm.at[src_v.at[j]], xr_v.at[j], sg[j])
                   for j in js]
            scs = []
            for n, j in enumerate(js):
                wait_ea(c0 + j, j)
                gds[n].wait()
                compute(j)
                scs.append(pltpu.async_copy(
                    ea_v.at[j], acc_sh.at[dst_v.at[j]], ssc[j], add=True))
            for n, j in enumerate(js):
                scs[n].wait()
                if j in prefetch_js:
                    start_idx(c0 + K + j, j)
                    start_ea(c0 + K + j, j)

        zero = jnp.zeros((LANES,), jnp.float32)

        def zrow(r, carry):
            for q in range(D // LANES):
                xr_v[0, r, pl.ds(q * LANES, LANES)] = zero
            return carry

        lax.fori_loop(0, C, zrow, 0)
        base = sid * rpt
        nfull = rpt // C
        rem = rpt - nfull * C
        for k in range(nfull):
            pltpu.sync_copy(xr_v.at[0], acc_sh.at[pl.ds(base + k * C, C)])
        if rem:
            pltpu.sync_copy(xr_v.at[0].at[pl.ds(0, rem)],
                            acc_sh.at[pl.ds(base + nfull * C, rem)])
        plsc.subcore_barrier()

        all_js = tuple(range(K))
        tail_js = tuple(range(ntail))
        for j in all_js:
            start_idx(j, j)
            start_ea(j, j)
        def loop_body(i, carry):
            iter_body(i * K, all_js, all_js)
            return carry
        lax.fori_loop(0, niter - 1, loop_body, 0)
        iter_body((niter - 1) * K, all_js, tail_js)
        if ntail:
            iter_body(niter * K, tail_js, ())

        plsc.subcore_barrier()

        pltpu.sync_copy(acc_sh.at[pl.ds(sid * rpt, rpt)],
                        out_hbm.at[cid, pl.ds(sid * rpt, rpt)])

    return sc_agg


def _tc_ffn_body(x_ref, a_ref, w1_ref, b1_ref, w2_ref, b2_ref, o_ref):
    xb = x_ref[...]
    h = xb + a_ref[0] + a_ref[1]
    h = jnp.dot(h, w1_ref[...], preferred_element_type=jnp.float32) + b1_ref[...]
    h = jnp.maximum(h, 0.0)
    h = jnp.dot(h, w2_ref[...], preferred_element_type=jnp.float32) + b2_ref[...]
    o_ref[...] = xb + jnp.maximum(h, 0.0)


def _tc_ffn(x, aggs, w1t, b1, w2t, b2, rows):
    N, D = x.shape
    grid = (N // rows,)
    row_spec = pl.BlockSpec((rows, D), lambda i: (i, 0))
    agg_spec = pl.BlockSpec((NC, rows, D), lambda i: (0, i, 0))
    full_spec = pl.BlockSpec((D, D), lambda i: (0, 0))
    vec_spec = pl.BlockSpec((1, D), lambda i: (0, 0))
    return pl.pallas_call(
        _tc_ffn_body,
        grid=grid,
        in_specs=[row_spec, agg_spec,
                  full_spec, vec_spec, full_spec, vec_spec],
        out_specs=row_spec,
        out_shape=jax.ShapeDtypeStruct((N, D), jnp.float32),
    )(x, aggs, w1t, b1, w2t, b2)


def kernel(x, edge_index, edge_attr, W1, b1, W2, b2):
    N, D = x.shape
    E = edge_index.shape[1]
    src = edge_index[0]
    dst = edge_index[1]
    sc_agg = _build_sc_agg(N, E, D, C=40)
    aggs = sc_agg(x, src, dst, edge_attr)
    return _tc_ffn(x, aggs, W1.T, b1.reshape(1, D),
                   W2.T, b2.reshape(1, D), rows=1000)


# speedup vs baseline: 6.9642x; 1.1014x over previous
import functools

import jax
import jax.numpy as jnp
from jax import lax
from jax.experimental import pallas as pl
from jax.experimental.pallas import tpu as pltpu
from jax.experimental.pallas import tpu_sc as plsc

NC = 2
NS = 16
LANES = 16
K = 3


def _build_sc_agg(N, E, D, C):
    NW = NC * NS
    epw = E // NW
    nchunk = epw // C
    assert (nchunk - 1) % K == 0 and nchunk >= 7
    rpt = (-(-N // NS) + 7) // 8 * 8
    n_pad = NS * rpt

    mesh = plsc.VectorSubcoreMesh(core_axis_name="c", subcore_axis_name="s")

    @functools.partial(
        pl.kernel,
        out_type=jax.ShapeDtypeStruct((NC, n_pad, D), jnp.float32),
        mesh=mesh,
        scratch_types=[
            pltpu.VMEM((K, C), jnp.int32),
            pltpu.VMEM((K, C), jnp.int32),
            pltpu.VMEM((K, C, D), jnp.float32),
            pltpu.VMEM((K, C, D), jnp.float32),
            pltpu.VMEM_SHARED((n_pad, D), jnp.float32),
            [pltpu.SemaphoreType.DMA] * K,
            [pltpu.SemaphoreType.DMA] * K,
            [pltpu.SemaphoreType.DMA] * K,
            [pltpu.SemaphoreType.DMA] * K,
            [pltpu.SemaphoreType.DMA] * K,
        ],
    )
    def sc_agg(x_hbm, src_hbm, dst_hbm, ea_hbm, out_hbm,
               src_v, dst_v, xr_v, ea_v, acc_sh,
               si, sd, se, sg, ssc):
        cid = lax.axis_index("c")
        sid = lax.axis_index("s")
        wid = cid * NS + sid
        ebase = wid * epw

        def start_idx(c, j):
            off = ebase + c * C
            pltpu.async_copy(src_hbm.at[pl.ds(off, C)], src_v.at[j], si[j])
            pltpu.async_copy(dst_hbm.at[pl.ds(off, C)], dst_v.at[j], sd[j])

        def wait_idx(c, j):
            off = ebase + c * C
            pltpu.make_async_copy(src_hbm.at[pl.ds(off, C)], src_v.at[j],
                                  si[j]).wait()
            pltpu.make_async_copy(dst_hbm.at[pl.ds(off, C)], dst_v.at[j],
                                  sd[j]).wait()

        def start_ea(c, j):
            off = ebase + c * C
            pltpu.async_copy(ea_hbm.at[pl.ds(off, C)], ea_v.at[j], se[j])

        def wait_ea(c, j):
            off = ebase + c * C
            pltpu.make_async_copy(ea_hbm.at[pl.ds(off, C)], ea_v.at[j],
                                  se[j]).wait()

        def compute(j):
            def vrow(r, carry):
                for u in range(4):
                    row = 4 * r + u
                    for q in range(D // LANES):
                        s = pl.ds(q * LANES, LANES)
                        ea_v[j, row, s] = jnp.maximum(
                            ea_v[j, row, s] + xr_v[j, row, s], 0.0)
                return carry
            lax.fori_loop(0, C // 4, vrow, 0)

        def wait_gather(b):
            pltpu.make_async_copy(x_hbm.at[src_v.at[b]], xr_v.at[b],
                                  sg[b]).wait()

        def wait_scatter(b):
            pltpu.make_async_copy(ea_v.at[b], acc_sh.at[dst_v.at[b]],
                                  ssc[b]).wait()

        def chunk_step(c, b, prev_scatter=True, next_gather=True,
                       prefetch=True):
            b1 = (b + 1) % K
            b2 = (b + 2) % K
            if next_gather:
                wait_idx(c + 1, b1)
                pltpu.async_copy(x_hbm.at[src_v.at[b1]], xr_v.at[b1], sg[b1])
            wait_ea(c, b)
            wait_gather(b)
            compute(b)
            if prev_scatter:
                wait_scatter(b2)
            if prefetch:
                start_idx(c + 2, b2)
                start_ea(c + 2, b2)
            pltpu.async_copy(ea_v.at[b], acc_sh.at[dst_v.at[b]], ssc[b],
                             add=True)

        zero = jnp.zeros((LANES,), jnp.float32)

        def zrow(r, carry):
            for q in range(D // LANES):
                ea_v[0, r, pl.ds(q * LANES, LANES)] = zero
            return carry

        lax.fori_loop(0, C, zrow, 0)
        base = sid * rpt
        nfull = rpt // C
        rem = rpt - nfull * C
        for k in range(nfull):
            pltpu.sync_copy(ea_v.at[0], acc_sh.at[pl.ds(base + k * C, C)])
        if rem:
            pltpu.sync_copy(ea_v.at[0].at[pl.ds(0, rem)],
                            acc_sh.at[pl.ds(base + nfull * C, rem)])
        plsc.subcore_barrier()

        start_idx(0, 0)
        start_ea(0, 0)
        start_idx(1, 1)
        start_ea(1, 1)
        wait_idx(0, 0)
        pltpu.async_copy(x_hbm.at[src_v.at[0]], xr_v.at[0], sg[0])
        chunk_step(0, 0, prev_scatter=False)
        def loop_body(i, carry):
            c0 = 1 + i * K
            for j in range(K):
                chunk_step(c0 + j, (1 + j) % K)
            return carry
        lax.fori_loop(0, (nchunk - 1 - K) // K, loop_body, 0)
        chunk_step(nchunk - 3, (nchunk - 3) % K)
        chunk_step(nchunk - 2, (nchunk - 2) % K, prefetch=False)
        chunk_step(nchunk - 1, (nchunk - 1) % K, next_gather=False,
                   prefetch=False)
        wait_scatter((nchunk - 1) % K)

        plsc.subcore_barrier()

        pltpu.sync_copy(acc_sh.at[pl.ds(sid * rpt, rpt)],
                        out_hbm.at[cid, pl.ds(sid * rpt, rpt)])

    return sc_agg


def _tc_ffn_body(x_ref, a_ref, w1_ref, b1_ref, w2_ref, b2_ref, o_ref):
    xb = x_ref[...]
    h = xb + a_ref[0] + a_ref[1]
    h = jnp.dot(h, w1_ref[...], preferred_element_type=jnp.float32) + b1_ref[...]
    h = jnp.maximum(h, 0.0)
    h = jnp.dot(h, w2_ref[...], preferred_element_type=jnp.float32) + b2_ref[...]
    o_ref[...] = xb + jnp.maximum(h, 0.0)


def _tc_ffn(x, aggs, w1t, b1, w2t, b2, rows):
    N, D = x.shape
    grid = (N // rows,)
    row_spec = pl.BlockSpec((rows, D), lambda i: (i, 0))
    agg_spec = pl.BlockSpec((NC, rows, D), lambda i: (0, i, 0))
    full_spec = pl.BlockSpec((D, D), lambda i: (0, 0))
    vec_spec = pl.BlockSpec((1, D), lambda i: (0, 0))
    return pl.pallas_call(
        _tc_ffn_body,
        grid=grid,
        in_specs=[row_spec, agg_spec,
                  full_spec, vec_spec, full_spec, vec_spec],
        out_specs=row_spec,
        out_shape=jax.ShapeDtypeStruct((N, D), jnp.float32),
    )(x, aggs, w1t, b1, w2t, b2)


def kernel(x, edge_index, edge_attr, W1, b1, W2, b2):
    N, D = x.shape
    E = edge_index.shape[1]
    src = edge_index[0]
    dst = edge_index[1]
    sc_agg = _build_sc_agg(N, E, D, C=40)
    aggs = sc_agg(x, src, dst, edge_attr)
    return _tc_ffn(x, aggs, W1.T, b1.reshape(1, D),
                   W2.T, b2.reshape(1, D), rows=2000)
